# baseline (speedup 1.0000x reference)
import functools

import jax
import jax.numpy as jnp
from jax import lax
from jax.experimental import pallas as pl
from jax.experimental.pallas import tpu as pltpu, tpu_sc as plsc

N = 16 * 1 * 512 * 512
K = int(N * 0.3)
NBIN = 1024
LOSS_MAX = 100.0
C_SCALE = float(NBIN) / LOSS_MAX
W_COARSE = LOSS_MAX / NBIN

LSTRIDE = NBIN + 17
NW = 32
PER_W = N // NW
CHUNK = 4096
NCHUNK = PER_W // CHUNK
GROUPS = CHUNK // 16

_MESH = plsc.VectorSubcoreMesh(core_axis_name="c", subcore_axis_name="s")


def _loss_body(pred_ref, gt_ref, loss_ref):
    p = pred_ref[...]
    g = gt_ref[...]
    log_p = jnp.maximum(jnp.log(p), -100.0)
    log_1mp = jnp.maximum(jnp.log(1.0 - p), -100.0)
    loss_ref[...] = -(g * log_p + (1.0 - g) * log_1mp)


def _bce_loss(pred2d, gt2d):
    rows = pred2d.shape[0]
    blk = rows // 8
    return pl.pallas_call(
        _loss_body,
        grid=(8,),
        in_specs=[pl.BlockSpec((blk, 512), lambda i: (i, 0)),
                  pl.BlockSpec((blk, 512), lambda i: (i, 0))],
        out_specs=pl.BlockSpec((blk, 512), lambda i: (i, 0)),
        out_shape=jax.ShapeDtypeStruct((rows, 512), jnp.float32),
    )(pred2d, gt2d)


def _hist_common(masked, loss_hbm, cc_hbm, cnt_hbm, sum_hbm,
                 buf0, buf1, hc, hs, oc, os_, sem0, sem1, ccv=None):
    wid = lax.axis_index("s") * 2 + lax.axis_index("c")
    row0 = wid * (PER_W // 512)

    if masked:
        pltpu.sync_copy(cc_hbm, ccv)

        @plsc.parallel_loop(0, NBIN // 16, unroll=2)
        def _redc(g):
            ac = ccv[pl.ds(g * 16, 16)]
            for w in range(1, NW):
                ac = ac + ccv[pl.ds(w * NBIN + g * 16, 16)]
            oc[pl.ds(g * 16, 16)] = ac

        def _scan(i, carry):
            run, nsel = carry
            g = NBIN // 16 - 1 - i
            gc = oc[pl.ds(g * 16, 16)]
            sfx = lax.rev(jnp.cumsum(lax.rev(gc, (0,))), (0,)) + run
            nsel = nsel + jnp.sum(jnp.where(sfx >= K, 1, 0))
            run = run + jnp.sum(gc)
            return run, nsel

        _, nsel = lax.fori_loop(0, NBIN // 16, _scan,
                                (jnp.int32(0), jnp.int32(0)))
        bsel_s = nsel - 1
        bsel = jnp.zeros((16,), jnp.int32) + bsel_s
        lo = (jnp.zeros((16,), jnp.float32)
              + bsel_s.astype(jnp.float32) * W_COARSE)
        invw = NBIN / W_COARSE

    zi = jnp.zeros((16,), jnp.int32)
    zf = jnp.zeros((16,), jnp.float32)

    @plsc.parallel_loop(0, (16 * LSTRIDE) // 16, unroll=8)
    def _zero(g):
        hc[pl.ds(g * 16, 16)] = zi
        hs[pl.ds(g * 16, 16)] = zf

    lane_off = lax.iota(jnp.int32, 16) * LSTRIDE
    ones_i = jnp.ones((16,), jnp.int32)

    bufs = (buf0, buf1)
    sems = (sem0, sem1)
    crows = CHUNK // 512
    pend = [None, None]
    pend[0] = pltpu.async_copy(loss_hbm.at[pl.ds(row0, crows), :], buf0, sem0)
    for c in range(NCHUNK):
        pend[c % 2].wait()
        if c + 1 < NCHUNK:
            pend[(c + 1) % 2] = pltpu.async_copy(
                loss_hbm.at[pl.ds(row0 + (c + 1) * crows, crows), :],
                bufs[(c + 1) % 2], sems[(c + 1) % 2])
        buf = bufs[c % 2]

        @plsc.parallel_loop(0, GROUPS, unroll=4)
        def _group(g):
            v = buf[lax.shift_right_logical(g, 5),
                    pl.ds(lax.bitwise_and(g, 31) * 16, 16)]
            if masked:
                cidx = jnp.clip((v * C_SCALE).astype(jnp.int32), 0, NBIN - 1)
                mask = cidx == bsel
                fidx = jnp.clip(((v - lo) * invw).astype(jnp.int32),
                                0, NBIN - 1)
                addr = fidx + lane_off
                plsc.addupdate_scatter(hc, [addr], ones_i, mask=mask)
                plsc.addupdate_scatter(hs, [addr], v, mask=mask)
            else:
                addr = jnp.clip((v * C_SCALE).astype(jnp.int32),
                                0, NBIN - 1) + lane_off
                plsc.addupdate_scatter(hc, [addr], ones_i)
                plsc.addupdate_scatter(hs, [addr], v)

    @plsc.parallel_loop(0, NBIN // 16, unroll=2)
    def _red(g):
        ac = hc[pl.ds(g * 16, 16)]
        af = hs[pl.ds(g * 16, 16)]
        for l in range(1, 16):
            ac = ac + hc[pl.ds(l * LSTRIDE + g * 16, 16)]
            af = af + hs[pl.ds(l * LSTRIDE + g * 16, 16)]
        oc[pl.ds(g * 16, 16)] = ac
        os_[pl.ds(g * 16, 16)] = af

    pltpu.sync_copy(oc, cnt_hbm.at[pl.ds(wid * NBIN, NBIN)])
    pltpu.sync_copy(os_, sum_hbm.at[pl.ds(wid * NBIN, NBIN)])


_SC_OUT = [jax.ShapeDtypeStruct((NW * NBIN,), jnp.int32),
           jax.ShapeDtypeStruct((NW * NBIN,), jnp.float32)]
_SC_SCRATCH = [
    pltpu.VMEM((CHUNK // 512, 512), jnp.float32),
    pltpu.VMEM((CHUNK // 512, 512), jnp.float32),
    pltpu.VMEM((16 * LSTRIDE,), jnp.int32),
    pltpu.VMEM((16 * LSTRIDE,), jnp.float32),
    pltpu.VMEM((NBIN,), jnp.int32),
    pltpu.VMEM((NBIN,), jnp.float32),
    pltpu.SemaphoreType.DMA,
    pltpu.SemaphoreType.DMA,
]


_SC_PARAMS = pltpu.CompilerParams(needs_layout_passes=False,
                                  use_tc_tiling_on_sc=True)


@functools.partial(
    pl.kernel,
    mesh=_MESH,
    compiler_params=_SC_PARAMS,
    out_type=_SC_OUT,
    scratch_types=_SC_SCRATCH,
)
def _sc_hist_coarse(loss_hbm, cnt_hbm, sum_hbm, *rest):
    _hist_common(False, loss_hbm, None, cnt_hbm, sum_hbm, *rest)


@functools.partial(
    pl.kernel,
    mesh=_MESH,
    compiler_params=_SC_PARAMS,
    out_type=_SC_OUT,
    scratch_types=_SC_SCRATCH + [pltpu.VMEM((NW * NBIN,), jnp.int32)],
)
def _sc_hist_fine(loss_hbm, cc_hbm, cnt_hbm, sum_hbm, *rest):
    _hist_common(True, loss_hbm, cc_hbm, cnt_hbm, sum_hbm, *rest)


def _suffix_sum(x):
    row = lax.broadcasted_iota(jnp.int32, (NBIN, NBIN), 0)
    col = lax.broadcasted_iota(jnp.int32, (NBIN, NBIN), 1)
    tri = (row >= col).astype(jnp.float32)
    return jnp.dot(x.reshape(1, NBIN), tri,
                   preferred_element_type=jnp.float32).reshape(NBIN)


def _final_body(cc_ref, cs_ref, fcnt_ref, fsum_ref, out_ref):
    c = jnp.sum(cc_ref[...], axis=0)
    s = jnp.sum(cs_ref[...], axis=0)
    cg = _suffix_sum(c.astype(jnp.float32))
    bsel = jnp.sum((cg >= K).astype(jnp.int32)) - 1
    bins = lax.iota(jnp.int32, NBIN)
    above = bins > bsel
    c_above = jnp.sum(jnp.where(above, c, 0)).astype(jnp.float32)
    s_above = jnp.sum(jnp.where(above, s, 0.0))
    total = jnp.sum(s)
    lo = bsel.astype(jnp.float32) * W_COARSE

    fc = jnp.sum(fcnt_ref[...], axis=0)
    fs = jnp.sum(fsum_ref[...], axis=0)
    cgf = _suffix_sum(fc.astype(jnp.float32))
    fsel = jnp.sum((c_above + cgf >= K).astype(jnp.int32)) - 1
    fabove = bins > fsel
    n_above_f = jnp.sum(jnp.where(fabove, fc, 0)).astype(jnp.float32)
    s_above_f = jnp.sum(jnp.where(fabove, fs, 0.0))
    needed = K - c_above - n_above_f
    w_f = W_COARSE / NBIN
    t_est = lo + (fsel.astype(jnp.float32) + 0.5) * w_f
    topk_sum = s_above + s_above_f + needed * t_est
    loss_total = total / (N + 1e-12) + topk_sum / K
    out_ref[...] = jnp.full((1, 1), loss_total)


def _final(cc, cs, fc, fs):
    return pl.pallas_call(
        _final_body,
        out_shape=jax.ShapeDtypeStruct((1, 1), jnp.float32),
    )(cc, cs, fc, fs)


def kernel(pred, gt):
    pred2d = pred.reshape(8192, 512)
    gt2d = gt.reshape(8192, 512)
    loss = _bce_loss(pred2d, gt2d)

    cc, cs = _sc_hist_coarse(loss)
    fc, fs = _sc_hist_fine(loss, cc)
    out = _final(cc.reshape(NW, NBIN), cs.reshape(NW, NBIN),
                 fc.reshape(NW, NBIN), fs.reshape(NW, NBIN))
    return out[0, 0]


# speedup vs baseline: 1.0981x; 1.0981x over previous
import functools

import jax
import jax.numpy as jnp
from jax import lax
from jax.experimental import pallas as pl
from jax.experimental.pallas import tpu as pltpu, tpu_sc as plsc

N = 16 * 1 * 512 * 512
K = int(N * 0.3)
NBIN = 1024
LOSS_MAX = 100.0
C_SCALE = float(NBIN) / LOSS_MAX
W_COARSE = LOSS_MAX / NBIN

LSTRIDE = NBIN + 17
NW = 32
PER_W = N // NW
CHUNK = 8192
NCHUNK = PER_W // CHUNK
GROUPS = CHUNK // 16

_MESH = plsc.VectorSubcoreMesh(core_axis_name="c", subcore_axis_name="s")


def _loss_body(pred_ref, gt_ref, loss_ref):
    p = pred_ref[...]
    g = gt_ref[...]
    log_p = jnp.maximum(jnp.log(p), -100.0)
    log_1mp = jnp.maximum(jnp.log(1.0 - p), -100.0)
    loss_ref[...] = -(g * log_p + (1.0 - g) * log_1mp)


def _bce_loss(pred2d, gt2d):
    rows = pred2d.shape[0]
    blk = rows // 8
    return pl.pallas_call(
        _loss_body,
        grid=(8,),
        in_specs=[pl.BlockSpec((blk, 512), lambda i: (i, 0)),
                  pl.BlockSpec((blk, 512), lambda i: (i, 0))],
        out_specs=pl.BlockSpec((blk, 512), lambda i: (i, 0)),
        out_shape=jax.ShapeDtypeStruct((rows, 512), jnp.float32),
    )(pred2d, gt2d)


def _hist_common(masked, loss_hbm, cc_hbm, cnt_hbm, sum_hbm,
                 buf0, buf1, hc, hs, oc, os_, sem0, sem1, ccv=None):
    wid = lax.axis_index("s") * 2 + lax.axis_index("c")
    row0 = wid * (PER_W // 512)

    if masked:
        q = (NW * NBIN) // 4
        qsel = lax.rem(wid, 4)
        cps = []
        for j in range(4):
            r = qsel + j
            r = jnp.where(r >= 4, r - 4, r)
            off = r * q
            cps.append(pltpu.async_copy(cc_hbm.at[pl.ds(off, q)],
                                        ccv.at[pl.ds(off, q)], sem0))
        for cp in cps:
            cp.wait()

        @plsc.parallel_loop(0, NBIN // 16, unroll=2)
        def _redc(g):
            ac = ccv[pl.ds(g * 16, 16)]
            for w in range(1, NW):
                ac = ac + ccv[pl.ds(w * NBIN + g * 16, 16)]
            oc[pl.ds(g * 16, 16)] = ac

        def _scan(i, carry):
            run, nsel = carry
            g = NBIN // 16 - 1 - i
            gc = oc[pl.ds(g * 16, 16)]
            sfx = lax.rev(jnp.cumsum(lax.rev(gc, (0,))), (0,)) + run
            nsel = nsel + jnp.sum(jnp.where(sfx >= K, 1, 0))
            run = run + jnp.sum(gc)
            return run, nsel

        _, nsel = lax.fori_loop(0, NBIN // 16, _scan,
                                (jnp.int32(0), jnp.int32(0)))
        bsel_s = nsel - 1
        bsel = jnp.zeros((16,), jnp.int32) + bsel_s
        lo = (jnp.zeros((16,), jnp.float32)
              + bsel_s.astype(jnp.float32) * W_COARSE)
        invw = NBIN / W_COARSE

    zi = jnp.zeros((16,), jnp.int32)
    zf = jnp.zeros((16,), jnp.float32)

    @plsc.parallel_loop(0, (16 * LSTRIDE) // 16, unroll=8)
    def _zero(g):
        hc[pl.ds(g * 16, 16)] = zi
        hs[pl.ds(g * 16, 16)] = zf

    lane_off = lax.iota(jnp.int32, 16) * LSTRIDE
    ones_i = jnp.ones((16,), jnp.int32)

    bufs = (buf0, buf1)
    sems = (sem0, sem1)
    crows = CHUNK // 512
    pend = [None, None]
    pend[0] = pltpu.async_copy(loss_hbm.at[pl.ds(row0, crows), :], buf0, sem0)
    for c in range(NCHUNK):
        pend[c % 2].wait()
        if c + 1 < NCHUNK:
            pend[(c + 1) % 2] = pltpu.async_copy(
                loss_hbm.at[pl.ds(row0 + (c + 1) * crows, crows), :],
                bufs[(c + 1) % 2], sems[(c + 1) % 2])
        buf = bufs[c % 2]

        @plsc.parallel_loop(0, GROUPS, unroll=4)
        def _group(g):
            v = buf[lax.shift_right_logical(g, 5),
                    pl.ds(lax.bitwise_and(g, 31) * 16, 16)]
            if masked:
                cidx = jnp.clip((v * C_SCALE).astype(jnp.int32), 0, NBIN - 1)
                mask = cidx == bsel
                fidx = jnp.clip(((v - lo) * invw).astype(jnp.int32),
                                0, NBIN - 1)
                addr = fidx + lane_off
                plsc.addupdate_scatter(hc, [addr], ones_i, mask=mask)
                plsc.addupdate_scatter(hs, [addr], v, mask=mask)
            else:
                addr = jnp.clip((v * C_SCALE).astype(jnp.int32),
                                0, NBIN - 1) + lane_off
                plsc.addupdate_scatter(hc, [addr], ones_i)
                plsc.addupdate_scatter(hs, [addr], v)

    @plsc.parallel_loop(0, NBIN // 16, unroll=2)
    def _red(g):
        ac = hc[pl.ds(g * 16, 16)]
        af = hs[pl.ds(g * 16, 16)]
        for l in range(1, 16):
            ac = ac + hc[pl.ds(l * LSTRIDE + g * 16, 16)]
            af = af + hs[pl.ds(l * LSTRIDE + g * 16, 16)]
        oc[pl.ds(g * 16, 16)] = ac
        os_[pl.ds(g * 16, 16)] = af

    pltpu.sync_copy(oc, cnt_hbm.at[pl.ds(wid * NBIN, NBIN)])
    pltpu.sync_copy(os_, sum_hbm.at[pl.ds(wid * NBIN, NBIN)])


_SC_OUT = [jax.ShapeDtypeStruct((NW * NBIN,), jnp.int32),
           jax.ShapeDtypeStruct((NW * NBIN,), jnp.float32)]
_SC_SCRATCH = [
    pltpu.VMEM((CHUNK // 512, 512), jnp.float32),
    pltpu.VMEM((CHUNK // 512, 512), jnp.float32),
    pltpu.VMEM((16 * LSTRIDE,), jnp.int32),
    pltpu.VMEM((16 * LSTRIDE,), jnp.float32),
    pltpu.VMEM((NBIN,), jnp.int32),
    pltpu.VMEM((NBIN,), jnp.float32),
    pltpu.SemaphoreType.DMA,
    pltpu.SemaphoreType.DMA,
]


_SC_PARAMS = pltpu.CompilerParams(needs_layout_passes=False,
                                  use_tc_tiling_on_sc=True)


@functools.partial(
    pl.kernel,
    mesh=_MESH,
    compiler_params=_SC_PARAMS,
    out_type=_SC_OUT,
    scratch_types=_SC_SCRATCH,
)
def _sc_hist_coarse(loss_hbm, cnt_hbm, sum_hbm, *rest):
    _hist_common(False, loss_hbm, None, cnt_hbm, sum_hbm, *rest)


@functools.partial(
    pl.kernel,
    mesh=_MESH,
    compiler_params=_SC_PARAMS,
    out_type=_SC_OUT,
    scratch_types=_SC_SCRATCH + [pltpu.VMEM((NW * NBIN,), jnp.int32)],
)
def _sc_hist_fine(loss_hbm, cc_hbm, cnt_hbm, sum_hbm, *rest):
    _hist_common(True, loss_hbm, cc_hbm, cnt_hbm, sum_hbm, *rest)


def _suffix_sum(x):
    row = lax.broadcasted_iota(jnp.int32, (NBIN, NBIN), 0)
    col = lax.broadcasted_iota(jnp.int32, (NBIN, NBIN), 1)
    tri = (row >= col).astype(jnp.float32)
    return jnp.dot(x.reshape(1, NBIN), tri,
                   preferred_element_type=jnp.float32).reshape(NBIN)


def _final_body(cc_ref, cs_ref, fcnt_ref, fsum_ref, out_ref):
    c = jnp.sum(cc_ref[...], axis=0)
    s = jnp.sum(cs_ref[...], axis=0)
    cg = _suffix_sum(c.astype(jnp.float32))
    bsel = jnp.sum((cg >= K).astype(jnp.int32)) - 1
    bins = lax.iota(jnp.int32, NBIN)
    above = bins > bsel
    c_above = jnp.sum(jnp.where(above, c, 0)).astype(jnp.float32)
    s_above = jnp.sum(jnp.where(above, s, 0.0))
    total = jnp.sum(s)
    lo = bsel.astype(jnp.float32) * W_COARSE

    fc = jnp.sum(fcnt_ref[...], axis=0)
    fs = jnp.sum(fsum_ref[...], axis=0)
    cgf = _suffix_sum(fc.astype(jnp.float32))
    fsel = jnp.sum((c_above + cgf >= K).astype(jnp.int32)) - 1
    fabove = bins > fsel
    n_above_f = jnp.sum(jnp.where(fabove, fc, 0)).astype(jnp.float32)
    s_above_f = jnp.sum(jnp.where(fabove, fs, 0.0))
    needed = K - c_above - n_above_f
    w_f = W_COARSE / NBIN
    t_est = lo + (fsel.astype(jnp.float32) + 0.5) * w_f
    topk_sum = s_above + s_above_f + needed * t_est
    loss_total = total / (N + 1e-12) + topk_sum / K
    out_ref[...] = jnp.full((1, 1), loss_total)


def _final(cc, cs, fc, fs):
    return pl.pallas_call(
        _final_body,
        out_shape=jax.ShapeDtypeStruct((1, 1), jnp.float32),
    )(cc, cs, fc, fs)


def kernel(pred, gt):
    pred2d = pred.reshape(8192, 512)
    gt2d = gt.reshape(8192, 512)
    loss = _bce_loss(pred2d, gt2d)

    cc, cs = _sc_hist_coarse(loss)
    fc, fs = _sc_hist_fine(loss, cc)
    out = _final(cc.reshape(NW, NBIN), cs.reshape(NW, NBIN),
                 fc.reshape(NW, NBIN), fs.reshape(NW, NBIN))
    return out[0, 0]


# speedup vs baseline: 1.1888x; 1.0826x over previous
import functools

import jax
import jax.numpy as jnp
from jax import lax
from jax.experimental import pallas as pl
from jax.experimental.pallas import tpu as pltpu, tpu_sc as plsc

N = 16 * 1 * 512 * 512
K = int(N * 0.3)
NBIN = 1024
LOSS_MAX = 100.0
C_SCALE = float(NBIN) / LOSS_MAX
W_COARSE = LOSS_MAX / NBIN

LSTRIDE = NBIN + 17
NW = 32
PER_W = N // NW
CHUNK = 8192
NCHUNK = PER_W // CHUNK
GROUPS = CHUNK // 16

_MESH = plsc.VectorSubcoreMesh(core_axis_name="c", subcore_axis_name="s")


def _loss_body(pred_ref, gt_ref, loss_ref):
    p = pred_ref[...]
    g = gt_ref[...]
    log_p = jnp.maximum(jnp.log(p), -100.0)
    log_1mp = jnp.maximum(jnp.log(1.0 - p), -100.0)
    loss_ref[...] = -(g * log_p + (1.0 - g) * log_1mp)


def _bce_loss(pred2d, gt2d):
    rows = pred2d.shape[0]
    blk = rows // 8
    return pl.pallas_call(
        _loss_body,
        grid=(8,),
        in_specs=[pl.BlockSpec((blk, 512), lambda i: (i, 0)),
                  pl.BlockSpec((blk, 512), lambda i: (i, 0))],
        out_specs=pl.BlockSpec((blk, 512), lambda i: (i, 0)),
        out_shape=jax.ShapeDtypeStruct((rows, 512), jnp.float32),
    )(pred2d, gt2d)


def _hist_common(masked, loss_hbm, cc_hbm, cnt_hbm, sum_hbm,
                 buf0, buf1, hc, hs, oc, os_, sem0, sem1, ccv=None):
    wid = lax.axis_index("s") * 2 + lax.axis_index("c")
    row0 = wid * (PER_W // 512)

    if masked:
        q = (NW * NBIN) // 4
        qsel = lax.rem(wid, 4)
        cps = []
        for j in range(4):
            r = qsel + j
            r = jnp.where(r >= 4, r - 4, r)
            off = r * q
            cps.append(pltpu.async_copy(cc_hbm.at[pl.ds(off, q)],
                                        ccv.at[pl.ds(off, q)], sem0))
        for cp in cps:
            cp.wait()

        @plsc.parallel_loop(0, NBIN // 16, unroll=2)
        def _redc(g):
            ac = ccv[pl.ds(g * 16, 16)]
            for w in range(1, NW):
                ac = ac + ccv[pl.ds(w * NBIN + g * 16, 16)]
            oc[pl.ds(g * 16, 16)] = ac

        def _scan(i, carry):
            run, nsel = carry
            g = NBIN // 16 - 1 - i
            gc = oc[pl.ds(g * 16, 16)]
            sfx = lax.rev(jnp.cumsum(lax.rev(gc, (0,))), (0,)) + run
            nsel = nsel + jnp.sum(jnp.where(sfx >= K, 1, 0))
            run = run + jnp.sum(gc)
            return run, nsel

        _, nsel = lax.fori_loop(0, NBIN // 16, _scan,
                                (jnp.int32(0), jnp.int32(0)))
        bsel_s = nsel - 1
        bsel = jnp.zeros((16,), jnp.int32) + bsel_s
        lo = (jnp.zeros((16,), jnp.float32)
              + bsel_s.astype(jnp.float32) * W_COARSE)
        invw = NBIN / W_COARSE

    zi = jnp.zeros((16,), jnp.int32)
    zf = jnp.zeros((16,), jnp.float32)

    @plsc.parallel_loop(0, (16 * LSTRIDE) // 16, unroll=8)
    def _zero(g):
        hc[pl.ds(g * 16, 16)] = zi
        hs[pl.ds(g * 16, 16)] = zf

    lane_off = lax.iota(jnp.int32, 16) * LSTRIDE
    ones_i = jnp.ones((16,), jnp.int32)

    bufs = (buf0, buf1)
    sems = (sem0, sem1)
    crows = CHUNK // 512
    pend = [None, None]
    pend[0] = pltpu.async_copy(loss_hbm.at[pl.ds(row0, crows), :], buf0, sem0)
    for c in range(NCHUNK):
        pend[c % 2].wait()
        if c + 1 < NCHUNK:
            pend[(c + 1) % 2] = pltpu.async_copy(
                loss_hbm.at[pl.ds(row0 + (c + 1) * crows, crows), :],
                bufs[(c + 1) % 2], sems[(c + 1) % 2])
        buf = bufs[c % 2]

        @plsc.parallel_loop(0, GROUPS, unroll=4)
        def _group(g):
            v = buf[lax.shift_right_logical(g, 5),
                    pl.ds(lax.bitwise_and(g, 31) * 16, 16)]
            if masked:
                fr = ((v - lo) * invw).astype(jnp.int32)
                mask = (v >= lo) & (fr < NBIN)
                addr = fr + lane_off
                plsc.addupdate_scatter(hc, [addr], ones_i, mask=mask)
                plsc.addupdate_scatter(hs, [addr], v, mask=mask)
            else:
                addr = jnp.clip((v * C_SCALE).astype(jnp.int32),
                                0, NBIN - 1) + lane_off
                plsc.addupdate_scatter(hc, [addr], ones_i)
                plsc.addupdate_scatter(hs, [addr], v)

    @plsc.parallel_loop(0, NBIN // 16, unroll=2)
    def _red(g):
        ac = hc[pl.ds(g * 16, 16)]
        af = hs[pl.ds(g * 16, 16)]
        for l in range(1, 16):
            ac = ac + hc[pl.ds(l * LSTRIDE + g * 16, 16)]
            af = af + hs[pl.ds(l * LSTRIDE + g * 16, 16)]
        oc[pl.ds(g * 16, 16)] = ac
        os_[pl.ds(g * 16, 16)] = af

    pltpu.sync_copy(oc, cnt_hbm.at[pl.ds(wid * NBIN, NBIN)])
    pltpu.sync_copy(os_, sum_hbm.at[pl.ds(wid * NBIN, NBIN)])


_SC_OUT = [jax.ShapeDtypeStruct((NW * NBIN,), jnp.int32),
           jax.ShapeDtypeStruct((NW * NBIN,), jnp.float32)]
_SC_SCRATCH = [
    pltpu.VMEM((CHUNK // 512, 512), jnp.float32),
    pltpu.VMEM((CHUNK // 512, 512), jnp.float32),
    pltpu.VMEM((16 * LSTRIDE,), jnp.int32),
    pltpu.VMEM((16 * LSTRIDE,), jnp.float32),
    pltpu.VMEM((NBIN,), jnp.int32),
    pltpu.VMEM((NBIN,), jnp.float32),
    pltpu.SemaphoreType.DMA,
    pltpu.SemaphoreType.DMA,
]


_SC_PARAMS = pltpu.CompilerParams(needs_layout_passes=False,
                                  use_tc_tiling_on_sc=True)


@functools.partial(
    pl.kernel,
    mesh=_MESH,
    compiler_params=_SC_PARAMS,
    out_type=_SC_OUT,
    scratch_types=_SC_SCRATCH,
)
def _sc_hist_coarse(loss_hbm, cnt_hbm, sum_hbm, *rest):
    _hist_common(False, loss_hbm, None, cnt_hbm, sum_hbm, *rest)


@functools.partial(
    pl.kernel,
    mesh=_MESH,
    compiler_params=_SC_PARAMS,
    out_type=_SC_OUT,
    scratch_types=_SC_SCRATCH + [pltpu.VMEM((NW * NBIN,), jnp.int32)],
)
def _sc_hist_fine(loss_hbm, cc_hbm, cnt_hbm, sum_hbm, *rest):
    _hist_common(True, loss_hbm, cc_hbm, cnt_hbm, sum_hbm, *rest)


def _suffix_sum(x):
    row = lax.broadcasted_iota(jnp.int32, (NBIN, NBIN), 0)
    col = lax.broadcasted_iota(jnp.int32, (NBIN, NBIN), 1)
    tri = (row >= col).astype(jnp.float32)
    return jnp.dot(x.reshape(1, NBIN), tri,
                   preferred_element_type=jnp.float32).reshape(NBIN)


def _final_body(cc_ref, cs_ref, fcnt_ref, fsum_ref, out_ref):
    c = jnp.sum(cc_ref[...], axis=0)
    s = jnp.sum(cs_ref[...], axis=0)
    cg = _suffix_sum(c.astype(jnp.float32))
    bsel = jnp.sum((cg >= K).astype(jnp.int32)) - 1
    bins = lax.iota(jnp.int32, NBIN)
    above = bins > bsel
    c_above = jnp.sum(jnp.where(above, c, 0)).astype(jnp.float32)
    s_above = jnp.sum(jnp.where(above, s, 0.0))
    total = jnp.sum(s)
    lo = bsel.astype(jnp.float32) * W_COARSE

    fc = jnp.sum(fcnt_ref[...], axis=0)
    fs = jnp.sum(fsum_ref[...], axis=0)
    cgf = _suffix_sum(fc.astype(jnp.float32))
    fsel = jnp.sum((c_above + cgf >= K).astype(jnp.int32)) - 1
    fabove = bins > fsel
    n_above_f = jnp.sum(jnp.where(fabove, fc, 0)).astype(jnp.float32)
    s_above_f = jnp.sum(jnp.where(fabove, fs, 0.0))
    needed = K - c_above - n_above_f
    w_f = W_COARSE / NBIN
    t_est = lo + (fsel.astype(jnp.float32) + 0.5) * w_f
    topk_sum = s_above + s_above_f + needed * t_est
    loss_total = total / (N + 1e-12) + topk_sum / K
    out_ref[...] = jnp.full((1, 1), loss_total)


def _final(cc, cs, fc, fs):
    return pl.pallas_call(
        _final_body,
        out_shape=jax.ShapeDtypeStruct((1, 1), jnp.float32),
    )(cc, cs, fc, fs)


def kernel(pred, gt):
    pred2d = pred.reshape(8192, 512)
    gt2d = gt.reshape(8192, 512)
    loss = _bce_loss(pred2d, gt2d)

    cc, cs = _sc_hist_coarse(loss)
    fc, fs = _sc_hist_fine(loss, cc)
    out = _final(cc.reshape(NW, NBIN), cs.reshape(NW, NBIN),
                 fc.reshape(NW, NBIN), fs.reshape(NW, NBIN))
    return out[0, 0]
